# baseline (speedup 1.0000x reference)
import functools

import numpy as np

import jax
import jax.numpy as jnp
from jax import lax
from jax.experimental import pallas as pl
from jax.experimental.pallas import tpu as pltpu
from jax.experimental.pallas import tpu_sc as plsc

N = 10000
E = 320000
F = 128
C = 40
CP = 48
NC, NS = 2, 16
NW = NC * NS
AR = 10240
RPT = AR // NS
B1, NB1F, NB1S = 56, 180, 180
B2, NB2F, NB2S = 128, 116, 44

_PERM = np.empty(F, np.int32)
for _g in range(F // 32):
    for _i in range(16):
        _PERM[32 * _g + 2 * _i] = 32 * _g + _i
        _PERM[32 * _g + 2 * _i + 1] = 32 * _g + 16 + _i


@functools.cache
def _make_sc_aggregate(d, B, NBF, NBS, with_counts, depth=2,
                       bf16_table=False):
    mesh = plsc.VectorSubcoreMesh(core_axis_name="c", subcore_axis_name="s")
    NBM = max(NBF, NBS)

    out_type = [jax.ShapeDtypeStruct((NC * AR, d), jnp.float32)]
    scratch = [
        pltpu.VMEM_SHARED((AR, d), jnp.float32),
        pltpu.VMEM((NBM + depth, B), jnp.int32),
        pltpu.VMEM((NBM + depth, B), jnp.int32),
    ]
    tdt = jnp.bfloat16 if bf16_table else jnp.float32
    scratch += [pltpu.VMEM((B, d), tdt) for _ in range(depth)]
    if bf16_table:
        scratch += [pltpu.VMEM((B, d), jnp.float32) for _ in range(depth)]
    scratch += [pltpu.SemaphoreType.DMA for _ in range(2 * depth)]
    if with_counts:
        out_type.append(jax.ShapeDtypeStruct((NC * AR, 16), jnp.int16))
        scratch += [
            pltpu.VMEM_SHARED((AR, 16), jnp.int16),
            pltpu.VMEM((B, 16), jnp.int16),
        ]

    def body(table, src2, dst2, zeros_d, zeros16, ones16, *refs):
        if with_counts:
            cacc, onesv = refs[-2:]
            refs = refs[:-2]
            out, cnt_out, acc, srcl, dstl = refs[:5]
            rest = refs[5:]
        else:
            out, acc, srcl, dstl = refs[:4]
            rest = refs[4:]
        rows = rest[:depth]
        rest = rest[depth:]
        if bf16_table:
            rowsf = rest[:depth]
            rest = rest[depth:]
        sg = rest[:depth]
        ss = rest[depth:2 * depth]
        cid = lax.axis_index("c")
        sid = lax.axis_index("s")
        nb = jnp.where(cid == 0, NBF, NBS)
        wrow = jnp.where(cid == 0, sid * NBF, NS * NBF + sid * NBS)

        pltpu.sync_copy(src2.at[pl.ds(wrow, NBM + depth)], srcl)
        pltpu.sync_copy(dst2.at[pl.ds(wrow, NBM + depth)], dstl)

        pltpu.sync_copy(zeros_d.at[pl.ds(sid * RPT, RPT)],
                        acc.at[pl.ds(sid * RPT, RPT)])
        if with_counts:
            pltpu.sync_copy(zeros16.at[pl.ds(sid * RPT, RPT)],
                            cacc.at[pl.ds(sid * RPT, RPT)])
            pltpu.sync_copy(ones16, onesv)
        plsc.subcore_barrier()

        for m in range(depth):
            pltpu.async_copy(table.at[srcl.at[m]], rows[m], sg[m])

        def step(t, carry):
            j = depth * t
            descs = []
            for m in range(depth):
                pltpu.make_async_copy(table.at[srcl.at[j + m]], rows[m],
                                      sg[m]).wait()
                if bf16_table:
                    def conv_row(i, c, m=m):
                        for g in range(d // 32):
                            lo, hi = plsc.unpack(
                                rows[m][i, pl.ds(32 * g, 32)],
                                format=plsc.PackFormat.INTERLEAVED)
                            rowsf[m][i, pl.ds(32 * g, 16)] = lo
                            rowsf[m][i, pl.ds(32 * g + 16, 16)] = hi
                        return c
                    lax.fori_loop(0, B, conv_row, 0)
                    sbuf = rowsf[m]
                else:
                    sbuf = rows[m]
                descs.append(pltpu.async_copy(sbuf, acc.at[dstl.at[j + m]],
                                              ss[m], add=True))
                if with_counts:
                    descs.append(pltpu.async_copy(onesv,
                                                  cacc.at[dstl.at[j + m]],
                                                  ss[m], add=True))
            per = 2 if with_counts else 1
            for m in range(depth):
                for dsc in descs[m * per:(m + 1) * per]:
                    dsc.wait()
                pltpu.async_copy(table.at[srcl.at[j + depth + m]], rows[m],
                                 sg[m])
            return carry

        lax.fori_loop(0, nb // depth, step, 0)
        for m in range(depth):
            pltpu.make_async_copy(table.at[srcl.at[nb + m]], rows[m],
                                  sg[m]).wait()
        plsc.subcore_barrier()

        rbase = cid * AR + sid * RPT
        pltpu.sync_copy(acc.at[pl.ds(sid * RPT, RPT)],
                        out.at[pl.ds(rbase, RPT)])
        if with_counts:
            pltpu.sync_copy(cacc.at[pl.ds(sid * RPT, RPT)],
                            cnt_out.at[pl.ds(rbase, RPT)])

    return pl.kernel(body, out_type=tuple(out_type), mesh=mesh,
                     scratch_types=tuple(scratch),
                     compiler_params=pltpu.CompilerParams(
                         use_tc_tiling_on_sc=False,
                         needs_layout_passes=False))


def _tc_dense1(part, cntp, xpad, w1, b1, w2p):
    blk = 256
    grid = (AR // blk,)

    def body(part_ref, cnt_ref, x_ref, w1_ref, b1_ref, w2_ref, out_ref):
        z = part_ref[0] + part_ref[1] + x_ref[...]
        cnt = (cnt_ref[0, :, 0:1] + cnt_ref[1, :, 0:1]).astype(jnp.float32)
        zm = z / (cnt + 1.0)
        h = jnp.maximum(
            jnp.dot(zm, w1_ref[...], preferred_element_type=jnp.float32)
            + b1_ref[...], 0.0)
        out_ref[...] = jnp.dot(h, w2_ref[...],
                               preferred_element_type=jnp.float32)

    return pl.pallas_call(
        body,
        grid=grid,
        in_specs=[
            pl.BlockSpec((NC, blk, F), lambda i: (0, i, 0)),
            pl.BlockSpec((NC, blk, 16), lambda i: (0, i, 0)),
            pl.BlockSpec((blk, F), lambda i: (i, 0)),
            pl.BlockSpec((F, F), lambda i: (0, 0)),
            pl.BlockSpec((1, F), lambda i: (0, 0)),
            pl.BlockSpec((F, CP), lambda i: (0, 0)),
        ],
        out_specs=pl.BlockSpec((blk, CP), lambda i: (i, 0)),
        out_shape=jax.ShapeDtypeStruct((AR, CP), jnp.float32),
    )(part, cntp, xpad, w1, b1, w2p)


def _tc_dense2(qpart, ppad, cntp, b2):
    blk = 400
    grid = (N // blk,)

    def body(qp_ref, p_ref, cnt_ref, b2_ref, out_ref):
        q48 = qp_ref[0] + qp_ref[1] + p_ref[...]
        cnt = (cnt_ref[0, :, 0:1] + cnt_ref[1, :, 0:1]).astype(jnp.float32)
        q = q48[:, :C] / (cnt + 1.0) + b2_ref[...]
        m = jnp.max(q, axis=1, keepdims=True)
        e = jnp.exp(q - m)
        out_ref[...] = (q - m) - jnp.log(jnp.sum(e, axis=1, keepdims=True))

    return pl.pallas_call(
        body,
        grid=grid,
        in_specs=[
            pl.BlockSpec((NC, blk, CP), lambda i: (0, i, 0)),
            pl.BlockSpec((blk, CP), lambda i: (i, 0)),
            pl.BlockSpec((NC, blk, 16), lambda i: (0, i, 0)),
            pl.BlockSpec((1, C), lambda i: (0, 0)),
        ],
        out_specs=pl.BlockSpec((blk, C), lambda i: (i, 0)),
        out_shape=jax.ShapeDtypeStruct((N, C), jnp.float32),
    )(qpart, ppad, cntp, b2)


def kernel(x, edge_index, W1, b1, W2, b2):
    src = edge_index[0]
    dst = edge_index[1]
    def padded(a, b, nbf, nbs):
        rows = max(NS * (nbf + nbs),
                   NS * nbf + (NS - 1) * nbs + max(nbf, nbs)) + 2
        pad = jnp.full((rows * b - E,), N, dtype=jnp.int32)
        return jnp.concatenate([a, pad]).reshape(rows, b)

    srcp1, dstp1 = padded(src, B1, NB1F, NB1S), padded(dst, B1, NB1F, NB1S)
    srcp2, dstp2 = padded(src, B2, NB2F, NB2S), padded(dst, B2, NB2F, NB2S)

    xpad = jnp.zeros((AR, F), jnp.float32).at[:N].set(x)
    w2p = jnp.zeros((F, CP), jnp.float32).at[:, :C].set(W2)
    zF = jnp.zeros((AR, F), jnp.float32)
    z16 = jnp.zeros((AR, 16), jnp.int16)
    zCP = jnp.zeros((AR, CP), jnp.float32)
    ones16 = jnp.ones((B1, 16), jnp.int16)

    xbf = jnp.zeros((AR, F), jnp.bfloat16).at[:N].set(
        x[:, _PERM].astype(jnp.bfloat16))
    part, cntp = _make_sc_aggregate(F, B1, NB1F, NB1S, True,
                                    bf16_table=True)(
        xbf, srcp1, dstp1, zF, z16, ones16)
    part = part.reshape(NC, AR, F)
    cntp = cntp.reshape(NC, AR, 16)

    ppad = _tc_dense1(part, cntp, xpad, W1, b1.reshape(1, F), w2p)

    qpart, = _make_sc_aggregate(CP, B2, NB2F, NB2S, False, depth=4)(
        ppad, srcp2, dstp2, zCP, z16, ones16)
    qpart = qpart.reshape(NC, AR, CP)

    return _tc_dense2(qpart, ppad, cntp, b2.reshape(1, C))


# speedup vs baseline: 1.0256x; 1.0256x over previous
import functools

import numpy as np

import jax
import jax.numpy as jnp
from jax import lax
from jax.experimental import pallas as pl
from jax.experimental.pallas import tpu as pltpu
from jax.experimental.pallas import tpu_sc as plsc

N = 10000
E = 320000
F = 128
C = 40
CP = 48
NC, NS = 2, 16
NW = NC * NS
AR = 10240
RPT = AR // NS
B1, NB1F, NB1S = 56, 180, 180
B2, NB2F, NB2S = 128, 116, 44

_PERM = np.empty(F, np.int32)
for _g in range(F // 32):
    for _i in range(16):
        _PERM[32 * _g + 2 * _i] = 32 * _g + _i
        _PERM[32 * _g + 2 * _i + 1] = 32 * _g + 16 + _i


@functools.cache
def _make_sc_aggregate(d, B, NBF, NBS, with_counts, depth=2,
                       bf16_table=False):
    mesh = plsc.VectorSubcoreMesh(core_axis_name="c", subcore_axis_name="s")
    NBM = max(NBF, NBS)

    out_type = [jax.ShapeDtypeStruct((NC * AR, d), jnp.float32)]
    scratch = [
        pltpu.VMEM_SHARED((AR, d), jnp.float32),
        pltpu.VMEM((NBM + depth, B), jnp.int32),
        pltpu.VMEM((NBM + depth, B), jnp.int32),
    ]
    tdt = jnp.bfloat16 if bf16_table else jnp.float32
    scratch += [pltpu.VMEM((B, d), tdt) for _ in range(depth)]
    if bf16_table:
        scratch += [pltpu.VMEM((B, d), jnp.float32) for _ in range(depth)]
    scratch += [pltpu.SemaphoreType.DMA for _ in range(2 * depth)]
    if with_counts:
        out_type.append(jax.ShapeDtypeStruct((NC * AR, 16), jnp.int16))
        scratch += [
            pltpu.VMEM_SHARED((AR, 16), jnp.int16),
            pltpu.VMEM((B, 16), jnp.int16),
        ]

    def body(table, src2, dst2, zeros_d, zeros16, ones16, *refs):
        if with_counts:
            cacc, onesv = refs[-2:]
            refs = refs[:-2]
            out, cnt_out, acc, srcl, dstl = refs[:5]
            rest = refs[5:]
        else:
            out, acc, srcl, dstl = refs[:4]
            rest = refs[4:]
        rows = rest[:depth]
        rest = rest[depth:]
        if bf16_table:
            rowsf = rest[:depth]
            rest = rest[depth:]
        sg = rest[:depth]
        ss = rest[depth:2 * depth]
        cid = lax.axis_index("c")
        sid = lax.axis_index("s")
        nb = jnp.where(cid == 0, NBF, NBS)
        wrow = jnp.where(cid == 0, sid * NBF, NS * NBF + sid * NBS)

        pltpu.sync_copy(src2.at[pl.ds(wrow, NBM + depth)], srcl)
        pltpu.sync_copy(dst2.at[pl.ds(wrow, NBM + depth)], dstl)

        pltpu.sync_copy(zeros_d.at[pl.ds(sid * RPT, RPT)],
                        acc.at[pl.ds(sid * RPT, RPT)])
        if with_counts:
            pltpu.sync_copy(zeros16.at[pl.ds(sid * RPT, RPT)],
                            cacc.at[pl.ds(sid * RPT, RPT)])
            pltpu.sync_copy(ones16, onesv)
        plsc.subcore_barrier()

        for m in range(depth):
            pltpu.async_copy(table.at[srcl.at[m]], rows[m], sg[m])

        def step(t, carry):
            j = depth * t
            descs = []
            for m in range(depth):
                pltpu.make_async_copy(table.at[srcl.at[j + m]], rows[m],
                                      sg[m]).wait()
                if bf16_table:
                    def conv_row(i, c, m=m):
                        for g in range(d // 32):
                            lo, hi = plsc.unpack(
                                rows[m][i, pl.ds(32 * g, 32)],
                                format=plsc.PackFormat.INTERLEAVED)
                            rowsf[m][i, pl.ds(32 * g, 16)] = lo
                            rowsf[m][i, pl.ds(32 * g + 16, 16)] = hi
                        return c
                    lax.fori_loop(0, B, conv_row, 0)
                    sbuf = rowsf[m]
                else:
                    sbuf = rows[m]
                descs.append(pltpu.async_copy(sbuf, acc.at[dstl.at[j + m]],
                                              ss[m], add=True))
                if with_counts:
                    descs.append(pltpu.async_copy(onesv,
                                                  cacc.at[dstl.at[j + m]],
                                                  ss[m], add=True))
            per = 2 if with_counts else 1
            for m in range(depth):
                for dsc in descs[m * per:(m + 1) * per]:
                    dsc.wait()
                pltpu.async_copy(table.at[srcl.at[j + depth + m]], rows[m],
                                 sg[m])
            return carry

        lax.fori_loop(0, nb // depth, step, 0)
        for m in range(depth):
            pltpu.make_async_copy(table.at[srcl.at[nb + m]], rows[m],
                                  sg[m]).wait()
        plsc.subcore_barrier()

        rbase = cid * AR + sid * RPT
        pltpu.sync_copy(acc.at[pl.ds(sid * RPT, RPT)],
                        out.at[pl.ds(rbase, RPT)])
        if with_counts:
            pltpu.sync_copy(cacc.at[pl.ds(sid * RPT, RPT)],
                            cnt_out.at[pl.ds(rbase, RPT)])

    return pl.kernel(body, out_type=tuple(out_type), mesh=mesh,
                     scratch_types=tuple(scratch),
                     compiler_params=pltpu.CompilerParams(
                         use_tc_tiling_on_sc=False,
                         needs_layout_passes=False))


def _tc_dense1(part, cntp, xpad, w1, b1, w2p):
    blk = 256
    grid = (AR // blk,)

    def body(part_ref, cnt_ref, x_ref, w1_ref, b1_ref, w2_ref, out_ref):
        z = part_ref[0] + part_ref[1] + x_ref[...]
        cnt = (cnt_ref[0, :, 0:1] + cnt_ref[1, :, 0:1]).astype(jnp.float32)
        zm = z / (cnt + 1.0)
        h = jnp.maximum(
            jnp.dot(zm, w1_ref[...], preferred_element_type=jnp.float32)
            + b1_ref[...], 0.0)
        out_ref[...] = jnp.dot(h, w2_ref[...],
                               preferred_element_type=jnp.float32)

    return pl.pallas_call(
        body,
        grid=grid,
        in_specs=[
            pl.BlockSpec((NC, blk, F), lambda i: (0, i, 0)),
            pl.BlockSpec((NC, blk, 16), lambda i: (0, i, 0)),
            pl.BlockSpec((blk, F), lambda i: (i, 0)),
            pl.BlockSpec((F, F), lambda i: (0, 0)),
            pl.BlockSpec((1, F), lambda i: (0, 0)),
            pl.BlockSpec((F, CP), lambda i: (0, 0)),
        ],
        out_specs=pl.BlockSpec((blk, CP), lambda i: (i, 0)),
        out_shape=jax.ShapeDtypeStruct((AR, CP), jnp.float32),
    )(part, cntp, xpad, w1, b1, w2p)


def _tc_dense2(qpart, ppad, cntp, b2):
    blk = 400
    grid = (N // blk,)

    def body(qp_ref, p_ref, cnt_ref, b2_ref, out_ref):
        q48 = qp_ref[0] + qp_ref[1] + p_ref[...]
        cnt = (cnt_ref[0, :, 0:1] + cnt_ref[1, :, 0:1]).astype(jnp.float32)
        q = q48[:, :C] / (cnt + 1.0) + b2_ref[...]
        m = jnp.max(q, axis=1, keepdims=True)
        e = jnp.exp(q - m)
        out_ref[...] = (q - m) - jnp.log(jnp.sum(e, axis=1, keepdims=True))

    return pl.pallas_call(
        body,
        grid=grid,
        in_specs=[
            pl.BlockSpec((NC, blk, CP), lambda i: (0, i, 0)),
            pl.BlockSpec((blk, CP), lambda i: (i, 0)),
            pl.BlockSpec((NC, blk, 16), lambda i: (0, i, 0)),
            pl.BlockSpec((1, C), lambda i: (0, 0)),
        ],
        out_specs=pl.BlockSpec((blk, C), lambda i: (i, 0)),
        out_shape=jax.ShapeDtypeStruct((N, C), jnp.float32),
    )(qpart, ppad, cntp, b2)


def kernel(x, edge_index, W1, b1, W2, b2):
    src = edge_index[0]
    dst = edge_index[1]
    def padded(a, b, nbf, nbs):
        rows = max(NS * (nbf + nbs),
                   NS * nbf + (NS - 1) * nbs + max(nbf, nbs)) + 2
        pad = jnp.full((rows * b - E,), N, dtype=jnp.int32)
        return jnp.concatenate([a, pad]).reshape(rows, b)

    srcp1, dstp1 = padded(src, B1, NB1F, NB1S), padded(dst, B1, NB1F, NB1S)
    srcp2, dstp2 = padded(src, B2, NB2F, NB2S), padded(dst, B2, NB2F, NB2S)

    xpad = jnp.zeros((AR, F), jnp.float32).at[:N].set(x)
    w2p = jnp.zeros((F, CP), jnp.float32).at[:, :C].set(W2)
    zF = jnp.zeros((AR, F), jnp.float32)
    z16 = jnp.zeros((AR, 16), jnp.int16)
    zCP = jnp.zeros((AR, CP), jnp.float32)
    ones16 = jnp.ones((B1, 16), jnp.int16)

    xbf = jnp.zeros((AR, F), jnp.bfloat16).at[:N].set(
        x[:, _PERM].astype(jnp.bfloat16))
    part, cntp = _make_sc_aggregate(F, B1, NB1F, NB1S, True,
                                    bf16_table=True)(
        xbf, srcp1, dstp1, zF, z16, ones16)
    part = part.reshape(NC, AR, F)
    cntp = cntp.reshape(NC, AR, 16)

    ppad = _tc_dense1(part, cntp, xpad, W1, b1.reshape(1, F), w2p)

    qpart, = _make_sc_aggregate(CP, B2, NB2F, NB2S, False)(
        ppad, srcp2, dstp2, zCP, z16, ones16)
    qpart = qpart.reshape(NC, AR, CP)

    return _tc_dense2(qpart, ppad, cntp, b2.reshape(1, C))
